# baseline (speedup 1.0000x reference)
import functools

import jax
import jax.numpy as jnp
from jax import lax
from jax.experimental import pallas as pl
from jax.experimental.pallas import tpu as pltpu
from jax.experimental.pallas import tpu_sc as plsc

B = 128
V = 100000
TOPP = 0.95
PEN_W = 10.0
SIL = (1049, 127, 1880, 1492, 972, 1031, 395, 2029, 581, 175, 1926, 407, 1316)

L = 16
NW = 32
ROWS_PER_W = B // NW

NB1 = 512
NB2 = 1024
LO = -26.0
W1 = 32.0 / NB1
W2 = W1 / NB2
INV_W1 = 1.0 / W1
INV_W2 = 1.0 / W2

GCH = 4000
NGC = V // GCH
VREGS_ROW = V // L
VREGS_GCH = GCH // L

BIG = 3.0e38
BIGI = 2**30


def _f1_of(xv):
    t = ((xv - LO) * INV_W1).astype(jnp.int32)
    return jnp.minimum(jnp.maximum(t, 0), NB1 - 1)


def _f2_of(xv, lo2):
    t = ((xv - lo2) * INV_W2).astype(jnp.int32)
    return jnp.minimum(jnp.maximum(t, 0), NB2 - 1)


def _body(x_hbm, g_hbm, out_hbm, xrow, hflat, gbuf0, gbuf1, tref,
          cand_x, cand_g, cand_i, resbuf, sem0, sem1):
    wid = lax.axis_index("s") * 2 + lax.axis_index("c")
    iota = lax.iota(jnp.int32, L)
    fzero = jnp.zeros((L,), jnp.float32)
    sil_mask = iota < len(SIL)
    sil_idx = jnp.zeros((L,), jnp.int32)
    for k, s in enumerate(SIL):
        sil_idx = jnp.where(iota == k, s, sil_idx)

    def zero_h(nwords):
        def z(i, c):
            hflat[pl.ds(i * L, L)] = fzero
            return c
        lax.fori_loop(0, nwords // L, z, 0)

    def bin_totals(nb):
        def tt(cb, c):
            acc = fzero
            for l in range(L):
                acc = acc + hflat[pl.ds(l * nb + cb * L, L)]
            tref[pl.ds(cb * L, L)] = acc
            return c
        lax.fori_loop(0, nb // L, tt, 0)

    def find_cross(nb, a0, p_c):
        def st(t, carry):
            acc, found, beta, a = carry
            cb = nb // L - 1 - t
            tv = tref[pl.ds(cb * L, L)]
            rv = lax.rev(tv, (0,))
            cs = plsc.cumsum(rv)
            m = (acc + cs) > p_c
            anyc = jnp.any(m)
            k = jnp.min(jnp.where(m, iota, L))
            ak = acc + jnp.sum(jnp.where(iota < k, rv, 0.0))
            betak = cb * L + (L - 1) - k
            use = anyc & jnp.logical_not(found)
            beta = jnp.where(use, betak, beta)
            a = jnp.where(use, ak, a)
            return (acc + jnp.sum(tv), found | anyc, beta, a)
        _, _, beta, a = lax.fori_loop(
            0, nb // L, st, (a0, False, jnp.int32(0), jnp.float32(0.0)))
        return beta, a

    def row_body(j, resvec):
        row = wid * ROWS_PER_W + j
        pltpu.sync_copy(x_hbm.at[row], xrow)
        plsc.addupdate_scatter(xrow, [sil_idx],
                               jnp.full((L,), -PEN_W, jnp.float32),
                               mask=sil_mask)
        zero_h(NB1 * L)

        def p1(i, c):
            xv = xrow[pl.ds(i * L, L)]
            ev = jnp.exp(xv)
            addr = iota * NB1 + _f1_of(xv)
            plsc.addupdate_scatter(hflat, [addr], ev)
            return c
        lax.fori_loop(0, VREGS_ROW, p1, 0, unroll=4)
        bin_totals(NB1)

        def csum(cb, acc):
            return acc + tref[pl.ds(cb * L, L)]
        c_total = jnp.sum(lax.fori_loop(0, NB1 // L, csum, fzero))
        p_c = jnp.float32(TOPP) * c_total
        beta1, a1 = find_cross(NB1, jnp.float32(0.0), p_c)
        lo2 = jnp.float32(LO) + beta1.astype(jnp.float32) * jnp.float32(W1)

        zero_h(NB2 * L)

        def p2(i, c):
            xv = xrow[pl.ds(i * L, L)]
            ev = jnp.exp(xv)
            inb = _f1_of(xv) == beta1
            addr = iota * NB2 + _f2_of(xv, lo2)
            plsc.addupdate_scatter(hflat, [addr], ev, mask=inb)
            return c
        lax.fori_loop(0, VREGS_ROW, p2, 0, unroll=4)
        bin_totals(NB2)
        beta2, a2 = find_cross(NB2, a1, p_c)

        def chunk_body(cbase, gbuf, carry):
            def p4(i, car):
                bestv, besti, cnt = car
                xv = xrow[pl.ds(cbase + i * L, L)]
                gv = gbuf[pl.ds(i * L, L)]
                f1 = _f1_of(xv)
                inb = f1 == beta1
                f2 = _f2_of(xv, lo2)
                above = (f1 > beta1) | (inb & (f2 > beta2))
                scv = jnp.where(above, xv + gv, -BIG)
                idxv = iota + (cbase + i * L)
                upd = scv > bestv
                bestv = jnp.where(upd, scv, bestv)
                besti = jnp.where(upd, idxv, besti)
                candm = inb & (f2 == beta2)

                @pl.when(jnp.any(candm))
                def _():
                    pos = jnp.minimum(
                        cnt + plsc.cumsum(candm.astype(jnp.int32)) - 1, 31)
                    plsc.store_scatter(cand_x, [pos], xv, mask=candm)
                    plsc.store_scatter(cand_g, [pos], gv, mask=candm)
                    plsc.store_scatter(cand_i, [pos], idxv, mask=candm)
                cnt = cnt + plsc.all_reduce_population_count(candm)
                return (bestv, besti, cnt)
            return lax.fori_loop(0, VREGS_GCH, p4, carry, unroll=2)

        carry = (jnp.full((L,), -BIG, jnp.float32), jnp.zeros((L,), jnp.int32),
                 jnp.zeros((L,), jnp.int32))
        gbufs = (gbuf0, gbuf1)
        sems = (sem0, sem1)
        desc = pltpu.async_copy(g_hbm.at[row, pl.ds(0, GCH)], gbuf0, sem0)
        for c in range(NGC):
            nxt = None
            if c + 1 < NGC:
                nxt = pltpu.async_copy(
                    g_hbm.at[row, pl.ds((c + 1) * GCH, GCH)],
                    gbufs[(c + 1) % 2], sems[(c + 1) % 2])
            desc.wait()
            carry = chunk_body(c * GCH, gbufs[c % 2], carry)
            desc = nxt
        bestv, besti, cnt = carry

        ma = jnp.max(bestv)
        ia = jnp.min(jnp.where(bestv == ma, besti, BIGI))
        ncand = jnp.max(cnt)

        cx = cand_x[pl.ds(0, L)]
        cg = cand_g[pl.ds(0, L)]
        ci = cand_i[pl.ds(0, L)]
        ce = jnp.exp(cx)
        sg = fzero
        eqle = jnp.zeros((L,), jnp.int32)
        for jj in range(L):
            validj = ncand > jj
            xj = cx[jj]
            ij = ci[jj]
            ej = ce[jj]
            sg = sg + jnp.where(validj & (xj > cx), ej, 0.0)
            eqle = eqle + jnp.where(validj & (xj == cx) & (ij <= ci), 1, 0)
        lane_valid = iota < ncand
        cv = a2 + sg + ce * eqle.astype(jnp.float32)
        ckept = lane_valid & ((cv - ce) <= p_c)
        cscore = jnp.where(ckept, cx + cg, -BIG)
        mc = jnp.max(cscore)
        ic = jnp.min(jnp.where(cscore == mc, ci, BIGI))
        use_c = (mc > ma) | ((mc == ma) & (ic < ia))
        res = jnp.where(use_c, ic, ia)
        return jnp.where(iota == j, res, resvec)

    resvec = lax.fori_loop(0, ROWS_PER_W, row_body, jnp.zeros((L,), jnp.int32))
    resbuf[pl.ds(0, L)] = resvec
    pltpu.sync_copy(resbuf, out_hbm.at[wid])


def _gumbel_const():
    skey = jax.random.fold_in(jax.random.key(0), 1)
    return jax.random.gumbel(skey, (B, V), jnp.float32)


@functools.cache
def _sc_call():
    mesh = plsc.VectorSubcoreMesh(core_axis_name="c", subcore_axis_name="s",
                                  num_cores=2, num_subcores=16)
    return pl.kernel(
        _body,
        out_type=jax.ShapeDtypeStruct((NW, L), jnp.int32),
        mesh=mesh,
        compiler_params=pltpu.CompilerParams(use_tc_tiling_on_sc=False,
                                             needs_layout_passes=False),
        scratch_types=[
            pltpu.VMEM((V,), jnp.float32),
            pltpu.VMEM((NB2 * L,), jnp.float32),
            pltpu.VMEM((GCH,), jnp.float32),
            pltpu.VMEM((GCH,), jnp.float32),
            pltpu.VMEM((NB2,), jnp.float32),
            pltpu.VMEM((32,), jnp.float32),
            pltpu.VMEM((32,), jnp.float32),
            pltpu.VMEM((32,), jnp.int32),
            pltpu.VMEM((L,), jnp.int32),
            pltpu.SemaphoreType.DMA,
            pltpu.SemaphoreType.DMA,
        ],
    )


def kernel(logits):
    assert logits.shape == (B, V) and logits.dtype == jnp.float32
    staging = _sc_call()(logits, _gumbel_const())
    return staging[:, :ROWS_PER_W].reshape(B, 1).astype(jnp.int32)


# speedup vs baseline: 2.5641x; 2.5641x over previous
import functools

import jax
import jax.numpy as jnp
from jax import lax
from jax.experimental import pallas as pl
from jax.experimental.pallas import tpu as pltpu
from jax.experimental.pallas import tpu_sc as plsc

B = 128
V = 100000
TOPP = 0.95
PEN_W = 10.0
SIL = (1049, 127, 1880, 1492, 972, 1031, 395, 2029, 581, 175, 1926, 407, 1316)

L = 16
NW = 32
ROWS_PER_W = B // NW

NB1 = 512
NB2 = 1024
LO = -26.0
W1 = 32.0 / NB1
W2 = W1 / NB2
INV_W1 = 1.0 / W1
INV_W2 = 1.0 / W2

GCH = 4000
NGC = V // GCH
VREGS_ROW = V // L
VREGS_GCH = GCH // L

BIG = 3.0e38
BIGI = 2**30


def _f1_of(xv):
    t = ((xv - LO) * INV_W1).astype(jnp.int32)
    return jnp.minimum(jnp.maximum(t, 0), NB1 - 1)


def _f2_of(xv, lo2):
    t = ((xv - lo2) * INV_W2).astype(jnp.int32)
    return jnp.minimum(jnp.maximum(t, 0), NB2 - 1)


def _body(x_hbm, g_hbm, out_hbm, xrow, hflat, gbuf0, gbuf1, tref,
          cand_x, cand_g, cand_i, resbuf, sem0, sem1):
    wid = lax.axis_index("s") * 2 + lax.axis_index("c")
    iota = lax.iota(jnp.int32, L)
    fzero = jnp.zeros((L,), jnp.float32)
    sil_mask = iota < len(SIL)
    sil_idx = jnp.zeros((L,), jnp.int32)
    for k, s in enumerate(SIL):
        sil_idx = jnp.where(iota == k, s, sil_idx)

    def zero_h(nwords):
        @plsc.parallel_loop(0, nwords // L, unroll=8)
        def _(i):
            hflat[pl.ds(i * L, L)] = fzero

    def bin_totals(nb):
        @plsc.parallel_loop(0, nb // L, unroll=2)
        def _(cb):
            acc = fzero
            for l in range(L):
                acc = acc + hflat[pl.ds(l * nb + cb * L, L)]
            tref[pl.ds(cb * L, L)] = acc

    def find_cross(nb, a0, p_c):
        def st(t, carry):
            acc, found, beta, a = carry
            cb = nb // L - 1 - t
            tv = tref[pl.ds(cb * L, L)]
            rv = lax.rev(tv, (0,))
            cs = plsc.cumsum(rv)
            m = (acc + cs) > p_c
            anyc = jnp.any(m)
            k = jnp.min(jnp.where(m, iota, L))
            ak = acc + jnp.sum(jnp.where(iota < k, rv, 0.0))
            betak = cb * L + (L - 1) - k
            use = anyc & jnp.logical_not(found)
            beta = jnp.where(use, betak, beta)
            a = jnp.where(use, ak, a)
            return (acc + jnp.sum(tv), found | anyc, beta, a)
        _, _, beta, a = lax.fori_loop(
            0, nb // L, st, (a0, False, jnp.int32(0), jnp.float32(0.0)))
        return beta, a

    def row_body(j, resvec):
        row = wid * ROWS_PER_W + j
        pltpu.sync_copy(x_hbm.at[row], xrow)
        plsc.addupdate_scatter(xrow, [sil_idx],
                               jnp.full((L,), -PEN_W, jnp.float32),
                               mask=sil_mask)
        zero_h(NB1 * L)

        @plsc.parallel_loop(0, VREGS_ROW, unroll=4)
        def _(i):
            xv = xrow[pl.ds(i * L, L)]
            ev = jnp.exp(xv)
            addr = iota * NB1 + _f1_of(xv)
            plsc.addupdate_scatter(hflat, [addr], ev)
        bin_totals(NB1)

        def csum(cb, acc):
            return acc + tref[pl.ds(cb * L, L)]
        c_total = jnp.sum(lax.fori_loop(0, NB1 // L, csum, fzero))
        p_c = jnp.float32(TOPP) * c_total
        beta1, a1 = find_cross(NB1, jnp.float32(0.0), p_c)
        lo2 = jnp.float32(LO) + beta1.astype(jnp.float32) * jnp.float32(W1)

        zero_h(NB2 * L)

        @plsc.parallel_loop(0, VREGS_ROW, unroll=4)
        def _(i):
            xv = xrow[pl.ds(i * L, L)]
            ev = jnp.exp(xv)
            inb = _f1_of(xv) == beta1
            addr = iota * NB2 + _f2_of(xv, lo2)
            plsc.addupdate_scatter(hflat, [addr], ev, mask=inb)
        bin_totals(NB2)
        beta2, a2 = find_cross(NB2, a1, p_c)

        def chunk_body(cbase, gbuf, carry):
            @plsc.parallel_loop(0, VREGS_GCH, unroll=4, carry=carry)
            def p4(i, car):
                bestv, besti, cnt = car
                xv = xrow[pl.ds(cbase + i * L, L)]
                gv = gbuf[pl.ds(i * L, L)]
                f1 = _f1_of(xv)
                inb = f1 == beta1
                f2 = _f2_of(xv, lo2)
                above = (f1 > beta1) | (inb & (f2 > beta2))
                scv = jnp.where(above, xv + gv, -BIG)
                idxv = iota + (cbase + i * L)
                upd = scv > bestv
                bestv = jnp.where(upd, scv, bestv)
                besti = jnp.where(upd, idxv, besti)
                candm = inb & (f2 == beta2)

                @pl.when(jnp.any(candm))
                def _():
                    pos = jnp.minimum(
                        cnt + plsc.cumsum(candm.astype(jnp.int32)) - 1, 31)
                    plsc.store_scatter(cand_x, [pos], xv, mask=candm)
                    plsc.store_scatter(cand_g, [pos], gv, mask=candm)
                    plsc.store_scatter(cand_i, [pos], idxv, mask=candm)
                cnt = cnt + plsc.all_reduce_population_count(candm)
                return (bestv, besti, cnt)
            return p4

        carry = (jnp.full((L,), -BIG, jnp.float32), jnp.zeros((L,), jnp.int32),
                 jnp.zeros((L,), jnp.int32))
        gbufs = (gbuf0, gbuf1)
        sems = (sem0, sem1)
        desc = pltpu.async_copy(g_hbm.at[row, pl.ds(0, GCH)], gbuf0, sem0)
        for c in range(NGC):
            nxt = None
            if c + 1 < NGC:
                nxt = pltpu.async_copy(
                    g_hbm.at[row, pl.ds((c + 1) * GCH, GCH)],
                    gbufs[(c + 1) % 2], sems[(c + 1) % 2])
            desc.wait()
            carry = chunk_body(c * GCH, gbufs[c % 2], carry)
            desc = nxt
        bestv, besti, cnt = carry

        ma = jnp.max(bestv)
        ia = jnp.min(jnp.where(bestv == ma, besti, BIGI))
        ncand = jnp.max(cnt)

        cx = cand_x[pl.ds(0, L)]
        cg = cand_g[pl.ds(0, L)]
        ci = cand_i[pl.ds(0, L)]
        ce = jnp.exp(cx)
        sg = fzero
        eqle = jnp.zeros((L,), jnp.int32)
        for jj in range(L):
            validj = ncand > jj
            xj = cx[jj]
            ij = ci[jj]
            ej = ce[jj]
            sg = sg + jnp.where(validj & (xj > cx), ej, 0.0)
            eqle = eqle + jnp.where(validj & (xj == cx) & (ij <= ci), 1, 0)
        lane_valid = iota < ncand
        cv = a2 + sg + ce * eqle.astype(jnp.float32)
        ckept = lane_valid & ((cv - ce) <= p_c)
        cscore = jnp.where(ckept, cx + cg, -BIG)
        mc = jnp.max(cscore)
        ic = jnp.min(jnp.where(cscore == mc, ci, BIGI))
        use_c = (mc > ma) | ((mc == ma) & (ic < ia))
        res = jnp.where(use_c, ic, ia)
        return jnp.where(iota == j, res, resvec)

    resvec = lax.fori_loop(0, ROWS_PER_W, row_body, jnp.zeros((L,), jnp.int32))
    resbuf[pl.ds(0, L)] = resvec
    pltpu.sync_copy(resbuf, out_hbm.at[wid])


def _gumbel_const():
    skey = jax.random.fold_in(jax.random.key(0), 1)
    return jax.random.gumbel(skey, (B, V), jnp.float32)


@functools.cache
def _sc_call():
    mesh = plsc.VectorSubcoreMesh(core_axis_name="c", subcore_axis_name="s",
                                  num_cores=2, num_subcores=16)
    return pl.kernel(
        _body,
        out_type=jax.ShapeDtypeStruct((NW, L), jnp.int32),
        mesh=mesh,
        compiler_params=pltpu.CompilerParams(use_tc_tiling_on_sc=False,
                                             needs_layout_passes=False),
        scratch_types=[
            pltpu.VMEM((V,), jnp.float32),
            pltpu.VMEM((NB2 * L,), jnp.float32),
            pltpu.VMEM((GCH,), jnp.float32),
            pltpu.VMEM((GCH,), jnp.float32),
            pltpu.VMEM((NB2,), jnp.float32),
            pltpu.VMEM((32,), jnp.float32),
            pltpu.VMEM((32,), jnp.float32),
            pltpu.VMEM((32,), jnp.int32),
            pltpu.VMEM((L,), jnp.int32),
            pltpu.SemaphoreType.DMA,
            pltpu.SemaphoreType.DMA,
        ],
    )


def kernel(logits):
    assert logits.shape == (B, V) and logits.dtype == jnp.float32
    staging = _sc_call()(logits, _gumbel_const())
    return staging[:, :ROWS_PER_W].reshape(B, 1).astype(jnp.int32)
